# baseline (speedup 1.0000x reference)
import functools

import jax
import jax.numpy as jnp
from jax import lax
from jax.experimental import pallas as pl
from jax.experimental.pallas import tpu as pltpu, tpu_sc as plsc

N = 10000
E = 320000
D_IN = 128
D_HID = 256
D_OUT = 128

NC = 2
NS = 16
NW = NC * NS
FP = 4

DEG_B = E // NW
DEG_BATCH = 2000
AGG_BATCH = 6400
AGG_NB = E // AGG_BATCH

_mesh = plsc.VectorSubcoreMesh(core_axis_name="c", subcore_axis_name="s")
_sc_params = pltpu.CompilerParams(needs_layout_passes=False)


def _deg_body(col_hbm, ew_hbm, out_hbm, c_v, w_v, deg_v):
    wid = lax.axis_index("s") * NC + lax.axis_index("c")

    def zero_body(i, _):
        deg_v[pl.ds(i * 16, 16)] = jnp.zeros((16,), jnp.float32)
        return ()

    lax.fori_loop(0, N // 16, zero_body, ())

    estart = wid * DEG_B

    def batch_body(b, _):
        base = estart + b * DEG_BATCH
        pltpu.sync_copy(col_hbm.at[pl.ds(base, DEG_BATCH)], c_v)
        pltpu.sync_copy(ew_hbm.at[pl.ds(base, DEG_BATCH)], w_v)

        def chunk_body(i, _):
            c = c_v[pl.ds(i * 16, 16)]
            w = w_v[pl.ds(i * 16, 16)]
            plsc.addupdate_scatter(deg_v, [c], w)
            return ()

        lax.fori_loop(0, DEG_BATCH // 16, chunk_body, ())
        return ()

    lax.fori_loop(0, DEG_B // DEG_BATCH, batch_body, ())
    pltpu.sync_copy(deg_v, out_hbm.at[wid])


_deg_call = pl.kernel(
    _deg_body,
    out_type=jax.ShapeDtypeStruct((NW, N), jnp.float32),
    mesh=_mesh,
    compiler_params=_sc_params,
    scratch_types=[
        pltpu.VMEM((DEG_BATCH,), jnp.int32),
        pltpu.VMEM((DEG_BATCH,), jnp.float32),
        pltpu.VMEM((N,), jnp.float32),
    ],
)


def _agg_body(f_total, gp_hbm, rc_hbm, ew_hbm, out_hbm, *scratch):
    NP = FP // 2
    gp_refs = scratch[:NP]
    acc_refs = scratch[NP:NP + FP]
    bufs0 = scratch[NP + FP:NP + FP + 2]
    bufs1 = scratch[NP + FP + 2:NP + FP + 4]
    sem0, sem1 = scratch[NP + FP + 4:]
    wid = lax.axis_index("s") * NC + lax.axis_index("c")
    n_pass = f_total // (NW * FP)
    B = AGG_BATCH

    def issue(bufs, sem, base):
        pltpu.async_copy(rc_hbm.at[pl.ds(base, B)], bufs[0], sem)
        pltpu.async_copy(ew_hbm.at[pl.ds(base, B)], bufs[1], sem)

    def drain(bufs, sem):
        pltpu.make_async_copy(rc_hbm.at[pl.ds(0, B)], bufs[0], sem).wait()
        pltpu.make_async_copy(ew_hbm.at[pl.ds(0, B)], bufs[1], sem).wait()

    def process(bufs):
        pp, ww = bufs

        @plsc.parallel_loop(0, B // 16, unroll=8)
        def _(i):
            rc = pp[pl.ds(i * 16, 16)]
            r = lax.shift_right_logical(rc, 14)
            c = jnp.bitwise_and(rc, 16383)
            w = ww[pl.ds(i * 16, 16)]
            for j in range(FP // 2):
                vi = plsc.load_gather(gp_refs[j], [r])
                flo = plsc.bitcast(lax.shift_left(vi, 16), jnp.float32)
                fhi = plsc.bitcast(
                    jnp.bitwise_and(vi, jnp.int32(-65536)), jnp.float32)
                plsc.addupdate_scatter(acc_refs[2 * j], [c], flo * w)
                plsc.addupdate_scatter(acc_refs[2 * j + 1], [c], fhi * w)

    for p in range(n_pass):
        fbase = wid * (f_total // NW) + p * FP
        for j in range(FP // 2):
            pltpu.sync_copy(gp_hbm.at[fbase // 2 + j], gp_refs[j])

        @plsc.parallel_loop(0, N // 16, unroll=5)
        def _(i):
            z = jnp.zeros((16,), jnp.float32)
            for f in range(FP):
                acc_refs[f][pl.ds(i * 16, 16)] = z

        issue(bufs0, sem0, 0)
        issue(bufs1, sem1, B)

        def gbody(g2, _):
            b0 = g2 * 2
            drain(bufs0, sem0)
            process(bufs0)
            issue(bufs0, sem0, ((b0 + 2) % AGG_NB) * B)
            drain(bufs1, sem1)
            process(bufs1)
            issue(bufs1, sem1, ((b0 + 3) % AGG_NB) * B)
            return ()

        lax.fori_loop(0, AGG_NB // 2, gbody, ())
        drain(bufs0, sem0)
        drain(bufs1, sem1)

        for f in range(FP):
            pltpu.sync_copy(acc_refs[f], out_hbm.at[fbase + f])


def _make_agg(f_total):
    return pl.kernel(
        functools.partial(_agg_body, f_total),
        out_type=jax.ShapeDtypeStruct((f_total, N), jnp.float32),
        mesh=_mesh,
        compiler_params=_sc_params,
        scratch_types=(
            [pltpu.VMEM((N,), jnp.int32) for _ in range(FP // 2)]
            + [pltpu.VMEM((N,), jnp.float32) for _ in range(FP)]
            + [
                pltpu.VMEM((AGG_BATCH,), jnp.int32),
                pltpu.VMEM((AGG_BATCH,), jnp.float32),
                pltpu.VMEM((AGG_BATCH,), jnp.int32),
                pltpu.VMEM((AGG_BATCH,), jnp.float32),
                pltpu.SemaphoreType.DMA,
                pltpu.SemaphoreType.DMA,
            ]
        ),
    )


_agg_hid = _make_agg(D_HID)
_agg_out = _make_agg(D_OUT)


def _pack_pairs(g):
    u = lax.bitcast_convert_type(g, jnp.uint32).reshape(g.shape[0] // 2, 2,
                                                        g.shape[1])
    lo = u[:, 0, :] + jnp.uint32(0x8000)
    hi = u[:, 1, :] + jnp.uint32(0x8000)
    packed = jnp.bitwise_or(jnp.bitwise_and(hi, jnp.uint32(0xFFFF0000)),
                            lax.shift_right_logical(lo, jnp.uint32(16)))
    return lax.bitcast_convert_type(packed, jnp.int32)


def _prep_body(degp_ref, x_ref, w1_ref, row_ref, col_ref,
               dis_ref, g1t_ref, g1p_ref, rc_ref):
    deg = jnp.sum(degp_ref[...], axis=0) + 1.0
    dis = lax.rsqrt(deg)
    dis_ref[...] = dis
    w1t = w1_ref[...].T
    h = lax.dot_general(w1t, x_ref[...], (((1,), (1,)), ((), ())),
                        preferred_element_type=jnp.float32)
    g1t = h * dis[None, :]
    g1t_ref[...] = g1t
    g1p_ref[...] = _pack_pairs(g1t)
    rc_ref[...] = jnp.bitwise_or(
        lax.shift_left(row_ref[...], 14), col_ref[...])


_prep_call = pl.pallas_call(
    _prep_body,
    out_shape=(
        jax.ShapeDtypeStruct((N,), jnp.float32),
        jax.ShapeDtypeStruct((D_HID, N), jnp.float32),
        jax.ShapeDtypeStruct((D_HID // 2, N), jnp.int32),
        jax.ShapeDtypeStruct((E,), jnp.int32),
    ),
)


def _mid_body(acc1t_ref, g1t_ref, dis_ref, w2_ref, b1_ref, g2t_ref, g2p_ref):
    dis = dis_ref[...]
    h2t = jnp.maximum(
        dis[None, :] * (acc1t_ref[...] + g1t_ref[...]) + b1_ref[...][:, None],
        0.0,
    )
    w2t = w2_ref[...].T
    g2t = lax.dot_general(w2t, h2t, (((1,), (0,)), ((), ())),
                          preferred_element_type=jnp.float32)
    g2t = g2t * dis[None, :]
    g2t_ref[...] = g2t
    g2p_ref[...] = _pack_pairs(g2t)


_mid_call = pl.pallas_call(
    _mid_body,
    out_shape=(
        jax.ShapeDtypeStruct((D_OUT, N), jnp.float32),
        jax.ShapeDtypeStruct((D_OUT // 2, N), jnp.int32),
    ),
)


def _final_body(acc2t_ref, g2t_ref, dis_ref, b2_ref, out_ref):
    dis = dis_ref[...]
    comb = dis[None, :] * (acc2t_ref[...] + g2t_ref[...]) + b2_ref[...][:, None]
    out_ref[...] = comb.T


_final_call = pl.pallas_call(
    _final_body,
    out_shape=jax.ShapeDtypeStruct((N, D_OUT), jnp.float32),
)


def kernel(x, edge_index, edge_weight, W1, b1, W2, b2):
    x = x.astype(jnp.float32)
    row = edge_index[0].astype(jnp.int32)
    col = edge_index[1].astype(jnp.int32)
    ew = edge_weight.astype(jnp.float32)

    degp = _deg_call(col, ew)
    dis, g1t, g1p, rc = _prep_call(degp, x, W1, row, col)
    acc1t = _agg_hid(g1p, rc, ew)
    g2t, g2p = _mid_call(acc1t, g1t, dis, W2, b1)
    acc2t = _agg_out(g2p, rc, ew)
    return _final_call(acc2t, g2t, dis, b2)


# speedup vs baseline: 1.0109x; 1.0109x over previous
import functools

import jax
import jax.numpy as jnp
from jax import lax
from jax.experimental import pallas as pl
from jax.experimental.pallas import tpu as pltpu, tpu_sc as plsc

N = 10000
E = 320000
D_IN = 128
D_HID = 256
D_OUT = 128

NC = 2
NS = 16
NW = NC * NS
FP = 4

DEG_B = E // NW
DEG_BATCH = 2000
AGG_BATCH = 3200
AGG_NB = E // AGG_BATCH

_mesh = plsc.VectorSubcoreMesh(core_axis_name="c", subcore_axis_name="s")
_sc_params = pltpu.CompilerParams(needs_layout_passes=False)


def _deg_body(col_hbm, ew_hbm, out_hbm, c_v, w_v, deg_v):
    wid = lax.axis_index("s") * NC + lax.axis_index("c")

    def zero_body(i, _):
        deg_v[pl.ds(i * 16, 16)] = jnp.zeros((16,), jnp.float32)
        return ()

    lax.fori_loop(0, N // 16, zero_body, ())

    estart = wid * DEG_B

    def batch_body(b, _):
        base = estart + b * DEG_BATCH
        pltpu.sync_copy(col_hbm.at[pl.ds(base, DEG_BATCH)], c_v)
        pltpu.sync_copy(ew_hbm.at[pl.ds(base, DEG_BATCH)], w_v)

        def chunk_body(i, _):
            c = c_v[pl.ds(i * 16, 16)]
            w = w_v[pl.ds(i * 16, 16)]
            plsc.addupdate_scatter(deg_v, [c], w)
            return ()

        lax.fori_loop(0, DEG_BATCH // 16, chunk_body, ())
        return ()

    lax.fori_loop(0, DEG_B // DEG_BATCH, batch_body, ())
    pltpu.sync_copy(deg_v, out_hbm.at[wid])


_deg_call = pl.kernel(
    _deg_body,
    out_type=jax.ShapeDtypeStruct((NW, N), jnp.float32),
    mesh=_mesh,
    compiler_params=_sc_params,
    scratch_types=[
        pltpu.VMEM((DEG_BATCH,), jnp.int32),
        pltpu.VMEM((DEG_BATCH,), jnp.float32),
        pltpu.VMEM((N,), jnp.float32),
    ],
)


def _agg_body(f_total, gp_hbm, rc_hbm, ew_hbm, out_hbm, *scratch):
    NP = FP // 2
    gp_refs = scratch[:NP]
    acc_refs = scratch[NP:NP + FP]
    bufs0 = scratch[NP + FP:NP + FP + 2]
    bufs1 = scratch[NP + FP + 2:NP + FP + 4]
    sem0, sem1 = scratch[NP + FP + 4:]
    wid = lax.axis_index("s") * NC + lax.axis_index("c")
    n_pass = f_total // (NW * FP)
    B = AGG_BATCH

    def issue(bufs, sem, base):
        pltpu.async_copy(rc_hbm.at[pl.ds(base, B)], bufs[0], sem)
        pltpu.async_copy(ew_hbm.at[pl.ds(base, B)], bufs[1], sem)

    def drain(bufs, sem):
        pltpu.make_async_copy(rc_hbm.at[pl.ds(0, B)], bufs[0], sem).wait()
        pltpu.make_async_copy(ew_hbm.at[pl.ds(0, B)], bufs[1], sem).wait()

    def process(bufs):
        pp, ww = bufs

        @plsc.parallel_loop(0, B // 16, unroll=8)
        def _(i):
            rc = pp[pl.ds(i * 16, 16)]
            r = lax.shift_right_logical(rc, 14)
            c = jnp.bitwise_and(rc, 16383)
            w = ww[pl.ds(i * 16, 16)]
            for j in range(FP // 2):
                vi = plsc.load_gather(gp_refs[j], [r])
                flo = plsc.bitcast(lax.shift_left(vi, 16), jnp.float32)
                fhi = plsc.bitcast(
                    jnp.bitwise_and(vi, jnp.int32(-65536)), jnp.float32)
                plsc.addupdate_scatter(acc_refs[2 * j], [c], flo * w)
                plsc.addupdate_scatter(acc_refs[2 * j + 1], [c], fhi * w)

    for p in range(n_pass):
        fbase = wid * (f_total // NW) + p * FP
        for j in range(FP // 2):
            pltpu.sync_copy(gp_hbm.at[fbase // 2 + j], gp_refs[j])

        @plsc.parallel_loop(0, N // 16, unroll=5)
        def _(i):
            z = jnp.zeros((16,), jnp.float32)
            for f in range(FP):
                acc_refs[f][pl.ds(i * 16, 16)] = z

        issue(bufs0, sem0, 0)
        issue(bufs1, sem1, B)

        def gbody(g2, _):
            b0 = g2 * 2
            drain(bufs0, sem0)
            process(bufs0)
            issue(bufs0, sem0, ((b0 + 2) % AGG_NB) * B)
            drain(bufs1, sem1)
            process(bufs1)
            issue(bufs1, sem1, ((b0 + 3) % AGG_NB) * B)
            return ()

        lax.fori_loop(0, AGG_NB // 2, gbody, ())
        drain(bufs0, sem0)
        drain(bufs1, sem1)

        for f in range(FP):
            pltpu.sync_copy(acc_refs[f], out_hbm.at[fbase + f])


def _make_agg(f_total):
    return pl.kernel(
        functools.partial(_agg_body, f_total),
        out_type=jax.ShapeDtypeStruct((f_total, N), jnp.float32),
        mesh=_mesh,
        compiler_params=_sc_params,
        scratch_types=(
            [pltpu.VMEM((N,), jnp.int32) for _ in range(FP // 2)]
            + [pltpu.VMEM((N,), jnp.float32) for _ in range(FP)]
            + [
                pltpu.VMEM((AGG_BATCH,), jnp.int32),
                pltpu.VMEM((AGG_BATCH,), jnp.float32),
                pltpu.VMEM((AGG_BATCH,), jnp.int32),
                pltpu.VMEM((AGG_BATCH,), jnp.float32),
                pltpu.SemaphoreType.DMA,
                pltpu.SemaphoreType.DMA,
            ]
        ),
    )


_agg_hid = _make_agg(D_HID)
_agg_out = _make_agg(D_OUT)


def _pack_pairs(g):
    u = lax.bitcast_convert_type(g, jnp.uint32).reshape(g.shape[0] // 2, 2,
                                                        g.shape[1])
    lo = u[:, 0, :] + jnp.uint32(0x8000)
    hi = u[:, 1, :] + jnp.uint32(0x8000)
    packed = jnp.bitwise_or(jnp.bitwise_and(hi, jnp.uint32(0xFFFF0000)),
                            lax.shift_right_logical(lo, jnp.uint32(16)))
    return lax.bitcast_convert_type(packed, jnp.int32)


def _prep_body(degp_ref, x_ref, w1_ref, row_ref, col_ref,
               dis_ref, g1t_ref, g1p_ref, rc_ref):
    deg = jnp.sum(degp_ref[...], axis=0) + 1.0
    dis = lax.rsqrt(deg)
    dis_ref[...] = dis
    w1t = w1_ref[...].T
    h = lax.dot_general(w1t, x_ref[...], (((1,), (1,)), ((), ())),
                        preferred_element_type=jnp.float32)
    g1t = h * dis[None, :]
    g1t_ref[...] = g1t
    g1p_ref[...] = _pack_pairs(g1t)
    rc_ref[...] = jnp.bitwise_or(
        lax.shift_left(row_ref[...], 14), col_ref[...])


_prep_call = pl.pallas_call(
    _prep_body,
    out_shape=(
        jax.ShapeDtypeStruct((N,), jnp.float32),
        jax.ShapeDtypeStruct((D_HID, N), jnp.float32),
        jax.ShapeDtypeStruct((D_HID // 2, N), jnp.int32),
        jax.ShapeDtypeStruct((E,), jnp.int32),
    ),
)


def _mid_body(acc1t_ref, g1t_ref, dis_ref, w2_ref, b1_ref, g2t_ref, g2p_ref):
    dis = dis_ref[...]
    h2t = jnp.maximum(
        dis[None, :] * (acc1t_ref[...] + g1t_ref[...]) + b1_ref[...][:, None],
        0.0,
    )
    w2t = w2_ref[...].T
    g2t = lax.dot_general(w2t, h2t, (((1,), (0,)), ((), ())),
                          preferred_element_type=jnp.float32)
    g2t = g2t * dis[None, :]
    g2t_ref[...] = g2t
    g2p_ref[...] = _pack_pairs(g2t)


_mid_call = pl.pallas_call(
    _mid_body,
    out_shape=(
        jax.ShapeDtypeStruct((D_OUT, N), jnp.float32),
        jax.ShapeDtypeStruct((D_OUT // 2, N), jnp.int32),
    ),
)


def _final_body(acc2t_ref, g2t_ref, dis_ref, b2_ref, out_ref):
    dis = dis_ref[...]
    comb = dis[None, :] * (acc2t_ref[...] + g2t_ref[...]) + b2_ref[...][:, None]
    out_ref[...] = comb.T


_final_call = pl.pallas_call(
    _final_body,
    out_shape=jax.ShapeDtypeStruct((N, D_OUT), jnp.float32),
)


def kernel(x, edge_index, edge_weight, W1, b1, W2, b2):
    x = x.astype(jnp.float32)
    row = edge_index[0].astype(jnp.int32)
    col = edge_index[1].astype(jnp.int32)
    ew = edge_weight.astype(jnp.float32)

    degp = _deg_call(col, ew)
    dis, g1t, g1p, rc = _prep_call(degp, x, W1, row, col)
    acc1t = _agg_hid(g1p, rc, ew)
    g2t, g2p = _mid_call(acc1t, g1t, dis, W2, b1)
    acc2t = _agg_out(g2p, rc, ew)
    return _final_call(acc2t, g2t, dis, b2)
